# baseline (speedup 1.0000x reference)
import dataclasses
import functools

import jax
import jax.numpy as jnp
from jax import lax
from jax.experimental import pallas as pl
from jax.experimental.pallas import tpu as pltpu
from jax.experimental.pallas import tpu_sc as plsc

_D = 1024
_H = 2048
_E = 8
_T = 2048
_K = 2
_N = _T * _K
_BT = 256
_CAP = _N + _E * _BT
_TILES = _CAP // _BT
_NSUB = 16
_NW = 2 * _NSUB
_TPW = _T // _NW
_PPW = _N // _NW
_SLICE = _N // _NSUB


def _sc_params():
    cp = pltpu.CompilerParams()
    if "needs_layout_passes" in pltpu.CompilerParams.__dataclass_fields__:
        cp = dataclasses.replace(cp, needs_layout_passes=False)
    return cp


def _router(x, wg):
    logits = x @ wg
    probs = jax.nn.softmax(logits, axis=-1)
    i1 = jnp.argmax(probs, axis=-1).astype(jnp.int32)
    p1 = jnp.max(probs, axis=-1)
    cols = lax.broadcasted_iota(jnp.int32, probs.shape, 1)
    masked = jnp.where(cols == i1[:, None], -1.0, probs)
    i2 = jnp.argmax(masked, axis=-1).astype(jnp.int32)
    p2 = jnp.max(masked, axis=-1)
    den = p1 + p2
    topi = jnp.stack([i1, i2], axis=1)
    gates = jnp.stack([p1 / den, p2 / den], axis=1)
    return topi, gates


def _route_dispatch(ef, x):
    mesh = plsc.VectorSubcoreMesh(core_axis_name="c", subcore_axis_name="s")

    @functools.partial(
        pl.kernel, mesh=mesh,
        out_type=(
            jax.ShapeDtypeStruct((_CAP, _D), jnp.float32),
            jax.ShapeDtypeStruct((2, _T), jnp.int32),
            jax.ShapeDtypeStruct((32,), jnp.int32),
        ),
        scratch_types=[
            pltpu.VMEM((_SLICE,), jnp.int32),
            pltpu.VMEM((16,), jnp.int32),
            pltpu.VMEM((16, 16), jnp.int32),
            pltpu.VMEM((_SLICE,), jnp.int32),
            pltpu.VMEM((_PPW,), jnp.int32),
            pltpu.VMEM((1, _TPW), jnp.int32),
            pltpu.VMEM((1, _TPW), jnp.int32),
            pltpu.VMEM((_TPW, _D), jnp.float32),
            pltpu.VMEM((32,), jnp.int32),
            pltpu.VMEM_SHARED((16, 16), jnp.int32),
            pltpu.VMEM_SHARED((_N,), jnp.int32),
        ],
        compiler_params=_sc_params(),
    )
    def k(ef_hbm, x_hbm, xs_hbm, pos2_hbm, te_hbm,
          ef_v, vec_v, hists_v, pos_v, myp_v, pk0_v, pk1_v, xrows_v, tev_v,
          hist_sh, pos_sh):
        c = lax.axis_index("c")
        s = lax.axis_index("s")
        wid = c * _NSUB + s
        iota = lax.iota(jnp.int32, 16)
        zero16 = iota * 0

        pltpu.sync_copy(ef_hbm.at[pl.ds(s * _SLICE, _SLICE)], ef_v)
        cnt = [jnp.int32(0)] * _E
        for i in range(_SLICE // 16):
            v = ef_v[pl.ds(i * 16, 16)]
            for e in range(_E):
                cnt[e] = cnt[e] + jnp.sum((v == e).astype(jnp.int32))
        histv = zero16
        for e in range(_E):
            histv = jnp.where(iota == e, cnt[e], histv)
        vec_v[...] = histv
        pltpu.sync_copy(vec_v, hist_sh.at[s])
        plsc.subcore_barrier()

        pltpu.sync_copy(hist_sh, hists_v)
        tot_vec = zero16
        mybase_vec = zero16
        for sp in range(_NSUB):
            hv = hists_v[sp]
            tot_vec = tot_vec + hv
            mybase_vec = mybase_vec + jnp.where(sp < s, hv, zero16)
        tot = [tot_vec[e] for e in range(_E)]
        mybase = [mybase_vec[e] for e in range(_E)]
        off = []
        tile_off = []
        run = jnp.int32(0)
        trun = jnp.int32(0)
        for e in range(_E):
            off.append(run)
            tile_off.append(trun)
            ntile = (tot[e] + (_BT - 1)) // _BT
            run = run + ntile * _BT
            trun = trun + ntile
        base = [off[e] + mybase[e] for e in range(_E)]

        run_e = [jnp.int32(0)] * _E
        for i in range(_SLICE // 16):
            v = ef_v[pl.ds(i * 16, 16)]
            posc = zero16
            for e in range(_E):
                m = v == e
                mi = m.astype(jnp.int32)
                r = plsc.cumsum(mi) + (base[e] + run_e[e] - 1)
                posc = jnp.where(m, r, posc)
                run_e[e] = run_e[e] + jnp.sum(mi)
            pos_v[pl.ds(i * 16, 16)] = posc
        pltpu.sync_copy(pos_v, pos_sh.at[pl.ds(s * _SLICE, _SLICE)])
        plsc.subcore_barrier()

        @pl.when(wid == 0)
        def _te():
            for j in range(2):
                tv = zero16 - 1
                for e in range(_E):
                    tv = tv + ((iota + 16 * j) >= tile_off[e]).astype(jnp.int32)
                tev_v[pl.ds(j * 16, 16)] = tv
            pltpu.sync_copy(tev_v, te_hbm)

        p0 = wid * _PPW
        t0 = wid * _TPW
        pltpu.sync_copy(pos_sh.at[pl.ds(p0, _PPW)], myp_v)
        for i in range(_TPW // 16):
            idx = iota * 2 + i * 32
            pk0_v[0, pl.ds(i * 16, 16)] = plsc.load_gather(myp_v, [idx])
            pk1_v[0, pl.ds(i * 16, 16)] = plsc.load_gather(myp_v, [idx + 1])
        pltpu.sync_copy(pk0_v.at[0], pos2_hbm.at[0, pl.ds(t0, _TPW)])
        pltpu.sync_copy(pk1_v.at[0], pos2_hbm.at[1, pl.ds(t0, _TPW)])
        pltpu.sync_copy(x_hbm.at[pl.ds(t0, _TPW)], xrows_v)
        pltpu.sync_copy(xrows_v, xs_hbm.at[pk0_v.at[0]])
        pltpu.sync_copy(xrows_v, xs_hbm.at[pk1_v.at[0]])

    return k(ef, x)


def _ffn_body(te_ref, xs_ref, w1_ref, b1_ref, w2_ref, b2_ref, ys_ref):
    xb = xs_ref[...].astype(jnp.bfloat16)
    w1 = w1_ref[0].astype(jnp.bfloat16)
    h = lax.dot_general(
        xb, w1, (((1,), (0,)), ((), ())),
        preferred_element_type=jnp.float32) + b1_ref[0]
    h = jax.nn.gelu(h)
    w2 = w2_ref[0].astype(jnp.bfloat16)
    ys_ref[...] = lax.dot_general(
        h.astype(jnp.bfloat16), w2, (((1,), (0,)), ((), ())),
        preferred_element_type=jnp.float32) + b2_ref[0]


def _ffn(te, xs, w1, b1, w2, b2):
    grid_spec = pltpu.PrefetchScalarGridSpec(
        num_scalar_prefetch=1,
        grid=(_TILES,),
        in_specs=[
            pl.BlockSpec((_BT, _D), lambda i, te: (i, 0)),
            pl.BlockSpec((1, _D, _H), lambda i, te: (te[i], 0, 0)),
            pl.BlockSpec((1, 1, _H), lambda i, te: (te[i], 0, 0)),
            pl.BlockSpec((1, _H, _D), lambda i, te: (te[i], 0, 0)),
            pl.BlockSpec((1, 1, _D), lambda i, te: (te[i], 0, 0)),
        ],
        out_specs=pl.BlockSpec((_BT, _D), lambda i, te: (i, 0)),
    )
    return pl.pallas_call(
        _ffn_body,
        grid_spec=grid_spec,
        out_shape=jax.ShapeDtypeStruct((_CAP, _D), jnp.float32),
    )(te, xs, w1, b1.reshape(_E, 1, _H), w2, b2.reshape(_E, 1, _D))


def _gather_back(ys, pos2):
    mesh = plsc.VectorSubcoreMesh(core_axis_name="c", subcore_axis_name="s")

    @functools.partial(
        pl.kernel, mesh=mesh,
        out_type=(
            jax.ShapeDtypeStruct((_T, _D), jnp.float32),
            jax.ShapeDtypeStruct((_T, _D), jnp.float32),
        ),
        scratch_types=[
            pltpu.VMEM((1, _TPW), jnp.int32),
            pltpu.VMEM((_TPW, _D), jnp.float32),
        ],
        compiler_params=_sc_params(),
    )
    def k(ys_hbm, pos2_hbm, yk0_hbm, yk1_hbm, pk_v, rows_v):
        c = lax.axis_index("c")
        s = lax.axis_index("s")
        t0 = (c * _NSUB + s) * _TPW
        pltpu.sync_copy(pos2_hbm.at[0, pl.ds(t0, _TPW)], pk_v.at[0])
        pltpu.sync_copy(ys_hbm.at[pk_v.at[0]], rows_v)
        pltpu.sync_copy(rows_v, yk0_hbm.at[pl.ds(t0, _TPW)])
        pltpu.sync_copy(pos2_hbm.at[1, pl.ds(t0, _TPW)], pk_v.at[0])
        pltpu.sync_copy(ys_hbm.at[pk_v.at[0]], rows_v)
        pltpu.sync_copy(rows_v, yk1_hbm.at[pl.ds(t0, _TPW)])

    return k(ys, pos2)


def _combine_body(g_ref, y0_ref, y1_ref, o_ref):
    g = g_ref[...]
    o_ref[...] = g[:, 0:1] * y0_ref[...] + g[:, 1:2] * y1_ref[...]


def _combine(gates, yk0, yk1):
    bc = 512
    return pl.pallas_call(
        _combine_body,
        grid=(_T // bc,),
        in_specs=[
            pl.BlockSpec((bc, _K), lambda t: (t, 0)),
            pl.BlockSpec((bc, _D), lambda t: (t, 0)),
            pl.BlockSpec((bc, _D), lambda t: (t, 0)),
        ],
        out_specs=pl.BlockSpec((bc, _D), lambda t: (t, 0)),
        out_shape=jax.ShapeDtypeStruct((_T, _D), jnp.float32),
    )(gates, yk0, yk1)


def kernel(x, Wg, W1, b1, W2, b2):
    eids, gates = _router(x, Wg)
    ef = eids.reshape(_N)
    xs, pos2, te = _route_dispatch(ef, x)
    ys = _ffn(te, xs, W1, b1, W2, b2)
    yk0, yk1 = _gather_back(ys, pos2)
    return _combine(gates, yk0, yk1)


# speedup vs baseline: 1.0361x; 1.0361x over previous
import dataclasses
import functools

import jax
import jax.numpy as jnp
from jax import lax
from jax.experimental import pallas as pl
from jax.experimental.pallas import tpu as pltpu
from jax.experimental.pallas import tpu_sc as plsc

_D = 1024
_H = 2048
_E = 8
_T = 2048
_K = 2
_N = _T * _K
_BT = 512
_CAP = _N + _E * _BT
_TILES = _CAP // _BT
_NSUB = 16
_NW = 2 * _NSUB
_TPW = _T // _NW
_PPW = _N // _NW
_SLICE = _N // _NSUB


def _sc_params():
    cp = pltpu.CompilerParams()
    if "needs_layout_passes" in pltpu.CompilerParams.__dataclass_fields__:
        cp = dataclasses.replace(cp, needs_layout_passes=False)
    return cp


def _router(x, wg):
    logits = x @ wg
    probs = jax.nn.softmax(logits, axis=-1)
    i1 = jnp.argmax(probs, axis=-1).astype(jnp.int32)
    p1 = jnp.max(probs, axis=-1)
    cols = lax.broadcasted_iota(jnp.int32, probs.shape, 1)
    masked = jnp.where(cols == i1[:, None], -1.0, probs)
    i2 = jnp.argmax(masked, axis=-1).astype(jnp.int32)
    p2 = jnp.max(masked, axis=-1)
    den = p1 + p2
    topi = jnp.stack([i1, i2], axis=1)
    gates = jnp.stack([p1 / den, p2 / den], axis=1)
    return topi, gates


def _route_dispatch(ef, x):
    mesh = plsc.VectorSubcoreMesh(core_axis_name="c", subcore_axis_name="s")

    @functools.partial(
        pl.kernel, mesh=mesh,
        out_type=(
            jax.ShapeDtypeStruct((_CAP, _D), jnp.float32),
            jax.ShapeDtypeStruct((2, _T), jnp.int32),
            jax.ShapeDtypeStruct((32,), jnp.int32),
        ),
        scratch_types=[
            pltpu.VMEM((_SLICE,), jnp.int32),
            pltpu.VMEM((16,), jnp.int32),
            pltpu.VMEM((16, 16), jnp.int32),
            pltpu.VMEM((_SLICE,), jnp.int32),
            pltpu.VMEM((_PPW,), jnp.int32),
            pltpu.VMEM((1, _TPW), jnp.int32),
            pltpu.VMEM((1, _TPW), jnp.int32),
            pltpu.VMEM((_TPW, _D), jnp.float32),
            pltpu.VMEM((32,), jnp.int32),
            pltpu.VMEM_SHARED((16, 16), jnp.int32),
            pltpu.VMEM_SHARED((_N,), jnp.int32),
        ],
        compiler_params=_sc_params(),
    )
    def k(ef_hbm, x_hbm, xs_hbm, pos2_hbm, te_hbm,
          ef_v, vec_v, hists_v, pos_v, myp_v, pk0_v, pk1_v, xrows_v, tev_v,
          hist_sh, pos_sh):
        c = lax.axis_index("c")
        s = lax.axis_index("s")
        wid = c * _NSUB + s
        iota = lax.iota(jnp.int32, 16)
        zero16 = iota * 0

        pltpu.sync_copy(ef_hbm.at[pl.ds(s * _SLICE, _SLICE)], ef_v)
        cnt = [jnp.int32(0)] * _E
        for i in range(_SLICE // 16):
            v = ef_v[pl.ds(i * 16, 16)]
            for e in range(_E):
                cnt[e] = cnt[e] + jnp.sum((v == e).astype(jnp.int32))
        histv = zero16
        for e in range(_E):
            histv = jnp.where(iota == e, cnt[e], histv)
        vec_v[...] = histv
        pltpu.sync_copy(vec_v, hist_sh.at[s])
        plsc.subcore_barrier()

        pltpu.sync_copy(hist_sh, hists_v)
        tot_vec = zero16
        mybase_vec = zero16
        for sp in range(_NSUB):
            hv = hists_v[sp]
            tot_vec = tot_vec + hv
            mybase_vec = mybase_vec + jnp.where(sp < s, hv, zero16)
        tot = [tot_vec[e] for e in range(_E)]
        mybase = [mybase_vec[e] for e in range(_E)]
        off = []
        tile_off = []
        run = jnp.int32(0)
        trun = jnp.int32(0)
        for e in range(_E):
            off.append(run)
            tile_off.append(trun)
            ntile = (tot[e] + (_BT - 1)) // _BT
            run = run + ntile * _BT
            trun = trun + ntile
        base = [off[e] + mybase[e] for e in range(_E)]

        run_e = [jnp.int32(0)] * _E
        for i in range(_SLICE // 16):
            v = ef_v[pl.ds(i * 16, 16)]
            posc = zero16
            for e in range(_E):
                m = v == e
                mi = m.astype(jnp.int32)
                r = plsc.cumsum(mi) + (base[e] + run_e[e] - 1)
                posc = jnp.where(m, r, posc)
                run_e[e] = run_e[e] + jnp.sum(mi)
            pos_v[pl.ds(i * 16, 16)] = posc
        pltpu.sync_copy(pos_v, pos_sh.at[pl.ds(s * _SLICE, _SLICE)])
        plsc.subcore_barrier()

        @pl.when(wid == 0)
        def _te():
            for j in range(2):
                tv = zero16 - 1
                for e in range(_E):
                    tv = tv + ((iota + 16 * j) >= tile_off[e]).astype(jnp.int32)
                tev_v[pl.ds(j * 16, 16)] = tv
            pltpu.sync_copy(tev_v, te_hbm)

        p0 = wid * _PPW
        t0 = wid * _TPW
        pltpu.sync_copy(pos_sh.at[pl.ds(p0, _PPW)], myp_v)
        for i in range(_TPW // 16):
            idx = iota * 2 + i * 32
            pk0_v[0, pl.ds(i * 16, 16)] = plsc.load_gather(myp_v, [idx])
            pk1_v[0, pl.ds(i * 16, 16)] = plsc.load_gather(myp_v, [idx + 1])
        pltpu.sync_copy(pk0_v.at[0], pos2_hbm.at[0, pl.ds(t0, _TPW)])
        pltpu.sync_copy(pk1_v.at[0], pos2_hbm.at[1, pl.ds(t0, _TPW)])
        pltpu.sync_copy(x_hbm.at[pl.ds(t0, _TPW)], xrows_v)
        pltpu.sync_copy(xrows_v, xs_hbm.at[pk0_v.at[0]])
        pltpu.sync_copy(xrows_v, xs_hbm.at[pk1_v.at[0]])

    return k(ef, x)


def _ffn_body(te_ref, xs_ref, w1_ref, b1_ref, w2_ref, b2_ref, ys_ref):
    xb = xs_ref[...].astype(jnp.bfloat16)
    w1 = w1_ref[0].astype(jnp.bfloat16)
    h = lax.dot_general(
        xb, w1, (((1,), (0,)), ((), ())),
        preferred_element_type=jnp.float32) + b1_ref[0]
    h = jax.nn.gelu(h)
    w2 = w2_ref[0].astype(jnp.bfloat16)
    ys_ref[...] = lax.dot_general(
        h.astype(jnp.bfloat16), w2, (((1,), (0,)), ((), ())),
        preferred_element_type=jnp.float32) + b2_ref[0]


def _ffn(te, xs, w1, b1, w2, b2):
    grid_spec = pltpu.PrefetchScalarGridSpec(
        num_scalar_prefetch=1,
        grid=(_TILES,),
        in_specs=[
            pl.BlockSpec((_BT, _D), lambda i, te: (i, 0)),
            pl.BlockSpec((1, _D, _H), lambda i, te: (te[i], 0, 0)),
            pl.BlockSpec((1, 1, _H), lambda i, te: (te[i], 0, 0)),
            pl.BlockSpec((1, _H, _D), lambda i, te: (te[i], 0, 0)),
            pl.BlockSpec((1, 1, _D), lambda i, te: (te[i], 0, 0)),
        ],
        out_specs=pl.BlockSpec((_BT, _D), lambda i, te: (i, 0)),
    )
    return pl.pallas_call(
        _ffn_body,
        grid_spec=grid_spec,
        out_shape=jax.ShapeDtypeStruct((_CAP, _D), jnp.float32),
    )(te, xs, w1, b1.reshape(_E, 1, _H), w2, b2.reshape(_E, 1, _D))


def _gather_back(ys, pos2):
    mesh = plsc.VectorSubcoreMesh(core_axis_name="c", subcore_axis_name="s")

    @functools.partial(
        pl.kernel, mesh=mesh,
        out_type=(
            jax.ShapeDtypeStruct((_T, _D), jnp.float32),
            jax.ShapeDtypeStruct((_T, _D), jnp.float32),
        ),
        scratch_types=[
            pltpu.VMEM((1, _TPW), jnp.int32),
            pltpu.VMEM((_TPW, _D), jnp.float32),
        ],
        compiler_params=_sc_params(),
    )
    def k(ys_hbm, pos2_hbm, yk0_hbm, yk1_hbm, pk_v, rows_v):
        c = lax.axis_index("c")
        s = lax.axis_index("s")
        t0 = (c * _NSUB + s) * _TPW
        pltpu.sync_copy(pos2_hbm.at[0, pl.ds(t0, _TPW)], pk_v.at[0])
        pltpu.sync_copy(ys_hbm.at[pk_v.at[0]], rows_v)
        pltpu.sync_copy(rows_v, yk0_hbm.at[pl.ds(t0, _TPW)])
        pltpu.sync_copy(pos2_hbm.at[1, pl.ds(t0, _TPW)], pk_v.at[0])
        pltpu.sync_copy(ys_hbm.at[pk_v.at[0]], rows_v)
        pltpu.sync_copy(rows_v, yk1_hbm.at[pl.ds(t0, _TPW)])

    return k(ys, pos2)


def _combine_body(g_ref, y0_ref, y1_ref, o_ref):
    g = g_ref[...]
    o_ref[...] = g[:, 0:1] * y0_ref[...] + g[:, 1:2] * y1_ref[...]


def _combine(gates, yk0, yk1):
    bc = 512
    return pl.pallas_call(
        _combine_body,
        grid=(_T // bc,),
        in_specs=[
            pl.BlockSpec((bc, _K), lambda t: (t, 0)),
            pl.BlockSpec((bc, _D), lambda t: (t, 0)),
            pl.BlockSpec((bc, _D), lambda t: (t, 0)),
        ],
        out_specs=pl.BlockSpec((bc, _D), lambda t: (t, 0)),
        out_shape=jax.ShapeDtypeStruct((_T, _D), jnp.float32),
    )(gates, yk0, yk1)


def kernel(x, Wg, W1, b1, W2, b2):
    eids, gates = _router(x, Wg)
    ef = eids.reshape(_N)
    xs, pos2, te = _route_dispatch(ef, x)
    ys = _ffn(te, xs, W1, b1, W2, b2)
    yk0, yk1 = _gather_back(ys, pos2)
    return _combine(gates, yk0, yk1)
